# baseline (speedup 1.0000x reference)
import functools
import jax
import jax.numpy as jnp
import numpy as np
from jax import lax
from jax.experimental import pallas as pl
from jax.experimental.pallas import tpu as pltpu

N_ENT = 50000
N_REL = 500
N_EDGE = 800000
INIT_EMB = 100
GC1_EMB = 150
EMB_DIM = 200
CHANNELS = 200
KSIZE = 5
BATCH = 128


def _bn(x, axes):
    m = jnp.mean(x, axis=axes, keepdims=True)
    v = jnp.var(x, axis=axes, keepdims=True)
    return (x - m) / jnp.sqrt(v + 1e-5)


def _gcn(inp, w, b, alpha_tab, rows, cols, rtype, n):
    alp = alpha_tab[rtype][:, 0]
    support = inp @ w
    out = jax.ops.segment_sum(alp[:, None] * support[cols], rows, num_segments=n)
    out = out + jax.ops.segment_sum(alp[:, None] * support[rows], cols, num_segments=n)
    return out + b


_EBLK = 2048


def _logits_body(x_ref, e_ref, o_ref):
    x = x_ref[...]
    e = e_ref[...]
    acc = jax.lax.dot_general(x, e, (((1,), (1,)), ((), ())),
                              preferred_element_type=jnp.float32)
    o_ref[...] = jax.nn.sigmoid(acc)


def _logits_pallas(x, e_all):
    n = e_all.shape[0]
    grid = (pl.cdiv(n, _EBLK),)
    return pl.pallas_call(
        _logits_body,
        grid=grid,
        in_specs=[
            pl.BlockSpec((BATCH, EMB_DIM), lambda i: (0, 0)),
            pl.BlockSpec((_EBLK, EMB_DIM), lambda i: (i, 0)),
        ],
        out_specs=pl.BlockSpec((BATCH, _EBLK), lambda i: (0, i)),
        out_shape=jax.ShapeDtypeStruct((BATCH, n), jnp.float32),
    )(x, e_all)


@jax.jit
def _impl(e1, rel, X, adj_edge_index, adj_rel_type, emb_e, gc1_w, gc1_b,
          gc1_alpha, gc2_w, gc2_b, gc2_alpha, emb_rel, conv_w, conv_b,
          fc_w, fc_b):
    rows = adj_edge_index[0]
    cols = adj_edge_index[1]
    emb_initial = emb_e[X]
    x = _gcn(emb_initial, gc1_w, gc1_b, gc1_alpha, rows, cols, adj_rel_type, N_ENT)
    x = jnp.tanh(_bn(x, 0))
    x = _bn(_gcn(x, gc2_w, gc2_b, gc2_alpha, rows, cols, adj_rel_type, N_ENT), 0)
    e_all = jnp.tanh(x)
    e1_emb = e_all[e1]
    rel_emb = emb_rel[rel]
    stacked = jnp.concatenate([e1_emb, rel_emb], axis=1)
    x = _bn(stacked, (0, 2))
    pad = KSIZE // 2
    x = lax.conv_general_dilated(x, conv_w, (1,), ((pad, pad),),
                                 dimension_numbers=('NCH', 'OIH', 'NCH'))
    x = x + conv_b[None, :, None]
    x = jax.nn.relu(_bn(x, (0, 2)))
    x = x.reshape(BATCH, -1)
    x = x @ fc_w + fc_b
    x = jax.nn.relu(_bn(x, 0))
    return _logits_pallas(x, e_all)


def kernel(e1, rel, X, adj_edge_index, adj_rel_type, emb_e, gc1_w, gc1_b,
           gc1_alpha, gc2_w, gc2_b, gc2_alpha, emb_rel, conv_w, conv_b,
           fc_w, fc_b):
    return _impl(e1, rel, X, adj_edge_index, adj_rel_type, emb_e, gc1_w,
                 gc1_b, gc1_alpha, gc2_w, gc2_b, gc2_alpha, emb_rel,
                 conv_w, conv_b, fc_w, fc_b)


# speedup vs baseline: 1.7940x; 1.7940x over previous
import functools
import jax
import jax.numpy as jnp
import numpy as np
from jax import lax
from jax.experimental import pallas as pl
from jax.experimental.pallas import tpu as pltpu
from jax.experimental.pallas import tpu_sc as plsc

N_ENT = 50000
N_REL = 500
N_EDGE = 800000
INIT_EMB = 100
GC1_EMB = 150
EMB_DIM = 200
CHANNELS = 200
KSIZE = 5
BATCH = 128

_NCORE = 2
_NSUB = 16
_NPAD = 51200
_EPT = N_EDGE // _NSUB
_W = 2000
_NWIN = _EPT // _W
_G = 128
_CAP = 4352


def _bn(x, axes):
    m = jnp.mean(x, axis=axes, keepdims=True)
    v = jnp.var(x, axis=axes, keepdims=True)
    return (x - m) / jnp.sqrt(v + 1e-5)


def _make_sc_gcn(dp, chunk, acc_rows, zrows, nzdma):
    nk = dp // 16
    nchunk_per_core = _NPAD // chunk // _NCORE
    zpt = acc_rows // _NSUB
    mesh = plsc.VectorSubcoreMesh(core_axis_name="c", subcore_axis_name="s")

    @functools.partial(
        pl.kernel,
        mesh=mesh,
        compiler_params=pltpu.CompilerParams(needs_layout_passes=False,
                                             use_tc_tiling_on_sc=False),
        out_type=jax.ShapeDtypeStruct((_NPAD, dp), jnp.float32),
        scratch_types=[
            pltpu.VMEM((_W,), jnp.int32),
            pltpu.VMEM((_W,), jnp.int32),
            pltpu.VMEM((_W,), jnp.int32),
            pltpu.VMEM((512,), jnp.float32),
            pltpu.VMEM((_CAP,), jnp.int32),
            pltpu.VMEM((_CAP,), jnp.int32),
            pltpu.VMEM((_CAP,), jnp.float32),
            pltpu.VMEM((_G, dp), jnp.float32),
            pltpu.VMEM((zrows, dp), jnp.float32),
            pltpu.VMEM_SHARED((acc_rows, dp), jnp.float32),
        ],
    )
    def sc_gcn(rows_hbm, cols_hbm, rt_hbm, atab_hbm, sup_hbm, out_hbm,
               ew_r, ew_c, ew_t, atab_v, src_lin, dst_lin, alp_lin,
               rows_buf, zbuf, acc):
        c = lax.axis_index("c")
        s = lax.axis_index("s")
        pltpu.sync_copy(atab_hbm, atab_v)

        zero16 = jnp.zeros((16,), jnp.float32)

        def zb_body(i, _):
            zbuf[i // nk, pl.ds((i % nk) * 16, 16)] = zero16
            return 0

        lax.fori_loop(0, zrows * nk, zb_body, 0)

        zero16i = jnp.zeros((16,), jnp.int32)

        def zsrc_body(i, _):
            src_lin[pl.ds(i * 16, 16)] = zero16i
            return 0

        lax.fori_loop(0, _CAP // 16, zsrc_body, 0)

        dump16 = jnp.full((16,), chunk, jnp.int32)
        ebase = s * _EPT

        def chunk_body(j, _):
            lo = (c * nchunk_per_core + j) * chunk
            hi = lo + chunk

            def zero_body(q, _):
                pltpu.sync_copy(zbuf, acc.at[pl.ds(s * zpt + q * zrows, zrows)])
                return 0

            lax.fori_loop(0, nzdma, zero_body, 0)
            plsc.subcore_barrier()

            def win_body(w, _):
                base = ebase + w * _W
                pltpu.sync_copy(rows_hbm.at[pl.ds(base, _W)], ew_r)
                pltpu.sync_copy(cols_hbm.at[pl.ds(base, _W)], ew_c)
                pltpu.sync_copy(rt_hbm.at[pl.ds(base, _W)], ew_t)

                def comp_body(i, cnt):
                    r16 = ew_r[pl.ds(i * 16, 16)]
                    c16 = ew_c[pl.ds(i * 16, 16)]
                    t16 = ew_t[pl.ds(i * 16, 16)]
                    a16 = plsc.load_gather(atab_v, [t16])
                    m1 = (r16 >= lo) & (r16 < hi)
                    plsc.store_compressed(src_lin.at[pl.ds(cnt, 16)], c16, mask=m1)
                    plsc.store_compressed(dst_lin.at[pl.ds(cnt, 16)], r16 - lo, mask=m1)
                    plsc.store_compressed(alp_lin.at[pl.ds(cnt, 16)], a16, mask=m1)
                    cnt = cnt + jnp.sum(m1.astype(jnp.int32))
                    m2 = (c16 >= lo) & (c16 < hi)
                    plsc.store_compressed(src_lin.at[pl.ds(cnt, 16)], r16, mask=m2)
                    plsc.store_compressed(dst_lin.at[pl.ds(cnt, 16)], c16 - lo, mask=m2)
                    plsc.store_compressed(alp_lin.at[pl.ds(cnt, 16)], a16, mask=m2)
                    cnt = cnt + jnp.sum(m2.astype(jnp.int32))
                    return cnt

                cnt = lax.fori_loop(0, _W // 16, comp_body, 0)
                for q in range(_G // 16):
                    dst_lin[pl.ds(cnt + q * 16, 16)] = dump16
                ngrp = lax.div(cnt + _G - 1, _G)

                def grp_body(g, _):
                    pltpu.sync_copy(sup_hbm.at[src_lin.at[pl.ds(g * _G, _G)]],
                                    rows_buf)

                    def scale_body(r, _):
                        a = plsc.load_gather(
                            alp_lin, [jnp.full((16,), g * _G + r, jnp.int32)])
                        for kk in range(nk):
                            rows_buf[r, pl.ds(kk * 16, 16)] = (
                                rows_buf[r, pl.ds(kk * 16, 16)] * a)
                        return 0

                    lax.fori_loop(0, _G, scale_body, 0)

                    def scat_body(q, _):
                        dst16 = dst_lin[pl.ds(g * _G + q * 16, 16)]
                        pltpu.sync_copy(rows_buf.at[pl.ds(q * 16, 16)],
                                        acc.at[dst16], add=True)
                        return 0

                    lax.fori_loop(0, _G // 16, scat_body, 0)
                    return 0

                lax.fori_loop(0, ngrp, grp_body, 0)
                return 0

            lax.fori_loop(0, _NWIN, win_body, 0)
            plsc.subcore_barrier()

            rpt = chunk // _NSUB
            pltpu.sync_copy(acc.at[pl.ds(s * rpt, rpt)],
                            out_hbm.at[pl.ds(lo + s * rpt, rpt)])
            plsc.subcore_barrier()
            return 0

        lax.fori_loop(0, nchunk_per_core, chunk_body, 0)

    return sc_gcn


_sc_gcn_160 = _make_sc_gcn(160, 6400, 6528, 136, 3)
_sc_gcn_208 = _make_sc_gcn(208, 3200, 3328, 104, 2)


_EBLK = 2048


def _logits_body(x_ref, e_ref, o_ref):
    x = x_ref[...]
    e = e_ref[...]
    acc = jax.lax.dot_general(x, e, (((1,), (1,)), ((), ())),
                              preferred_element_type=jnp.float32)
    o_ref[...] = jax.nn.sigmoid(acc)


def _logits_pallas(x, e_all):
    n = e_all.shape[0]
    grid = (pl.cdiv(n, _EBLK),)
    return pl.pallas_call(
        _logits_body,
        grid=grid,
        in_specs=[
            pl.BlockSpec((BATCH, EMB_DIM), lambda i: (0, 0)),
            pl.BlockSpec((_EBLK, EMB_DIM), lambda i: (i, 0)),
        ],
        out_specs=pl.BlockSpec((BATCH, _EBLK), lambda i: (0, i)),
        out_shape=jax.ShapeDtypeStruct((BATCH, n), jnp.float32),
    )(x, e_all)


@jax.jit
def _impl(e1, rel, X, adj_edge_index, adj_rel_type, emb_e, gc1_w, gc1_b,
          gc1_alpha, gc2_w, gc2_b, gc2_alpha, emb_rel, conv_w, conv_b,
          fc_w, fc_b):
    rows = adj_edge_index[0]
    cols = adj_edge_index[1]
    rtype = adj_rel_type.astype(jnp.int32)
    emb_initial = emb_e[X]

    atab1 = jnp.pad(gc1_alpha[:, 0], (0, 512 - (N_REL + 1)))
    atab2 = jnp.pad(gc2_alpha[:, 0], (0, 512 - (N_REL + 1)))

    sup1 = emb_initial @ jnp.pad(gc1_w, ((0, 0), (0, 160 - GC1_EMB)))
    g1 = _sc_gcn_160(rows, cols, rtype, atab1, sup1)
    x = g1[:N_ENT, :GC1_EMB] + gc1_b
    x = jnp.tanh(_bn(x, 0))

    sup2 = x @ jnp.pad(gc2_w, ((0, 0), (0, 208 - EMB_DIM)))
    g2 = _sc_gcn_208(rows, cols, rtype, atab2, sup2)
    x = _bn(g2[:N_ENT, :EMB_DIM] + gc2_b, 0)
    e_all = jnp.tanh(x)

    e1_emb = e_all[e1]
    rel_emb = emb_rel[rel]
    stacked = jnp.concatenate([e1_emb, rel_emb], axis=1)
    x = _bn(stacked, (0, 2))
    pad = KSIZE // 2
    x = lax.conv_general_dilated(x, conv_w, (1,), ((pad, pad),),
                                 dimension_numbers=('NCH', 'OIH', 'NCH'))
    x = x + conv_b[None, :, None]
    x = jax.nn.relu(_bn(x, (0, 2)))
    x = x.reshape(BATCH, -1)
    x = x @ fc_w + fc_b
    x = jax.nn.relu(_bn(x, 0))
    return _logits_pallas(x, e_all)


def kernel(e1, rel, X, adj_edge_index, adj_rel_type, emb_e, gc1_w, gc1_b,
           gc1_alpha, gc2_w, gc2_b, gc2_alpha, emb_rel, conv_w, conv_b,
           fc_w, fc_b):
    return _impl(e1, rel, X, adj_edge_index, adj_rel_type, emb_e, gc1_w,
                 gc1_b, gc1_alpha, gc2_w, gc2_b, gc2_alpha, emb_rel,
                 conv_w, conv_b, fc_w, fc_b)
